# baseline (speedup 1.0000x reference)
import functools
import numpy as np
import jax
import jax.numpy as jnp
from jax import lax
from jax.experimental import pallas as pl
from jax.experimental.pallas import tpu as pltpu
from jax.experimental.pallas import tpu_sc as plsc

_ALPHA = 4


def kernel(frames):
    B, T, C, H, W = frames.shape
    nsel = T // _ALPHA
    idx = [int(v) for v in np.linspace(0.0, T - 1, nsel).astype(np.int32)]
    unsel = [f for f in range(T) if f not in idx]

    info = plsc.get_sparse_core_info()
    NW = info.num_cores * info.num_subcores
    n_pure = B * len(unsel) * C // NW
    n_fused = B * nsel * C // NW
    n_units = n_pure + n_fused

    def static_lookup(table, i):
        v = jnp.int32(0)
        for j, t in enumerate(table):
            v = v + jnp.where(i == j, t, 0)
        return v

    mesh = plsc.VectorSubcoreMesh(core_axis_name="c", subcore_axis_name="s")

    @functools.partial(
        pl.kernel,
        mesh=mesh,
        out_type=[
            jax.ShapeDtypeStruct((B, nsel, C, H, W), frames.dtype),
            jax.ShapeDtypeStruct((B, T, C, H, W), frames.dtype),
        ],
        scratch_types=[
            pltpu.VMEM((H, W), frames.dtype),
            pltpu.VMEM((H, W), frames.dtype),
            pltpu.SemaphoreType.DMA,
            pltpu.SemaphoreType.DMA,
            pltpu.SemaphoreType.DMA,
            pltpu.SemaphoreType.DMA,
            pltpu.SemaphoreType.DMA,
            pltpu.SemaphoreType.DMA,
        ],
    )
    def pack_k(frames_hbm, slow_hbm, fast_hbm, buf0, buf1,
               in0, in1, fs0, fs1, sl0, sl1):
        wid = lax.axis_index("s") * info.num_cores + lax.axis_index("c")
        bufs, in_sems = (buf0, buf1), (in0, in1)
        fast_sems, slow_sems = (fs0, fs1), (sl0, sl1)

        def unit(i):
            if i < n_pure:
                u = wid * n_pure + i
                c = u % C
                fpos = (u // C) % len(unsel)
                b = u // (C * len(unsel))
                f = static_lookup(unsel, fpos)
                return (frames_hbm.at[b, f, c], fast_hbm.at[b, f, c], None)
            u = wid * n_fused + (i - n_pure)
            c = u % C
            s = (u // C) % nsel
            b = u // (C * nsel)
            f = static_lookup(idx, s)
            return (frames_hbm.at[b, f, c], fast_hbm.at[b, f, c],
                    slow_hbm.at[b, s, c])

        in_flight = [None] * n_units
        out_flight = [None] * n_units

        src0, _, _ = unit(0)
        in_flight[0] = pltpu.async_copy(src0, bufs[0], in_sems[0])
        for i in range(n_units):
            bi = i % 2
            if i >= 1:
                for cp in out_flight[i - 1]:
                    cp.wait()
            if i + 1 < n_units:
                src, _, _ = unit(i + 1)
                in_flight[i + 1] = pltpu.async_copy(
                    src, bufs[(i + 1) % 2], in_sems[(i + 1) % 2])
            in_flight[i].wait()
            _, fast_dst, slow_dst = unit(i)
            outs = [pltpu.async_copy(bufs[bi], fast_dst, fast_sems[bi])]
            if slow_dst is not None:
                outs.append(pltpu.async_copy(bufs[bi], slow_dst, slow_sems[bi]))
            out_flight[i] = outs
        for cp in out_flight[n_units - 1]:
            cp.wait()

    slow, fast = pack_k(frames)
    return (slow, fast)
